# Initial kernel scaffold; baseline (speedup 1.0000x reference)
import jax
import jax.numpy as jnp
from jax.experimental import pallas as pl


def kernel(static_feat, W1, b1, W2, b2):
    raise NotImplementedError("write your pallas kernel here")


# speedup vs baseline: 18.7182x; 18.7182x over previous
import functools

import jax
import jax.numpy as jnp
from jax import lax
from jax.experimental import pallas as pl
from jax.experimental.pallas import tpu as pltpu
from jax.experimental.pallas import tpu_sc as plsc

N = 4096
XD = 128
DIM = 64
K_EDGES = 65536
A1 = 0.1
A2 = 2.0

TOT = N * N
NB = 4096
NC = 2
NS = 16
NW = NC * NS
PER_W = TOT // NW
CH = 8192
BLK = 128

_PREC = lax.Precision.DEFAULT


def _nodevec_body(x_ref, w1_ref, b1_ref, w2_ref, b2_ref, v1_ref, v2_ref):
    x = x_ref[...]
    z1 = lax.dot_general(x, w1_ref[...], (((1,), (1,)), ((), ())),
                         precision=_PREC)
    z2 = lax.dot_general(x, w2_ref[...], (((1,), (1,)), ((), ())),
                         precision=_PREC)
    v1_ref[...] = jnp.tanh(A1 * (z1 + b1_ref[...]))
    v2_ref[...] = jnp.tanh(A1 * (z2 + b2_ref[...]))


_nodevec_call = pl.pallas_call(
    _nodevec_body,
    out_shape=(
        jax.ShapeDtypeStruct((N, DIM), jnp.float32),
        jax.ShapeDtypeStruct((N, DIM), jnp.float32),
    ),
)


def _scores_body(v1_ref, v2_ref, p_ref):
    s = lax.dot_general(v1_ref[...], v2_ref[...], (((1,), (1,)), ((), ())),
                        precision=_PREC)
    p_ref[...] = 1.0 / (1.0 + jnp.exp(-A2 * s))


_scores_call = pl.pallas_call(
    _scores_body,
    grid=(N // BLK,),
    in_specs=[
        pl.BlockSpec((BLK, DIM), lambda i: (i, 0)),
        pl.BlockSpec((N, DIM), lambda i: (0, 0)),
    ],
    out_specs=pl.BlockSpec((BLK, N), lambda i: (i, 0)),
    out_shape=jax.ShapeDtypeStruct((N, N), jnp.float32),
)


def _sc_hist_body(p_hbm, par_hbm, out_hbm, par_v, buf_v, hist16_v, merged_v):
    wid = lax.axis_index("s") * NC + lax.axis_index("c")
    base = wid * PER_W
    pltpu.sync_copy(par_hbm, par_v)
    lo = par_v[pl.ds(0, 16)]
    scale = par_v[pl.ds(16, 16)]
    mlo = par_v[pl.ds(32, 16)]
    mhi = par_v[pl.ds(48, 16)]

    def _zero(j, carry):
        hist16_v[pl.ds(j * 16, 16)] = jnp.zeros((16,), jnp.float32)
        return carry

    lax.fori_loop(0, (16 * NB) // 16, _zero, 0)

    ones = jnp.ones((16,), jnp.float32)
    lane_off = jax.lax.iota(jnp.int32, 16) * NB

    def _chunk(c, carry):
        pltpu.sync_copy(p_hbm.at[pl.ds(base + c * CH, CH)], buf_v)

        def _vec(j, inner):
            x = buf_v[pl.ds(j * 16, 16)]
            v = (x - lo) * scale
            i0 = v.astype(jnp.int32)
            iv = i0 - (i0.astype(jnp.float32) > v).astype(jnp.int32)
            iv = jnp.clip(iv, 0, NB - 1)
            m = (x >= mlo) & (x < mhi)
            plsc.addupdate_scatter(hist16_v, [lane_off + iv], ones, mask=m)
            return inner

        lax.fori_loop(0, CH // 16, _vec, 0)
        return carry

    lax.fori_loop(0, PER_W // CH, _chunk, 0)

    def _merge(j, carry):
        def _lane(l, a):
            return a + hist16_v[pl.ds(l * NB + j * 16, 16)]

        acc = lax.fori_loop(0, 16, _lane, jnp.zeros((16,), jnp.float32))
        merged_v[pl.ds(j * 16, 16)] = acc
        return carry

    lax.fori_loop(0, NB // 16, _merge, 0)
    pltpu.sync_copy(merged_v, out_hbm.at[pl.ds(wid * NB, NB)])


_sc_hist_call = functools.partial(
    pl.kernel,
    out_type=jax.ShapeDtypeStruct((NW * NB,), jnp.float32),
    mesh=plsc.VectorSubcoreMesh(core_axis_name="c", subcore_axis_name="s",
                                num_cores=NC, num_subcores=NS),
    scratch_types=[
        pltpu.VMEM((64,), jnp.float32),
        pltpu.VMEM((CH,), jnp.float32),
        pltpu.VMEM((16 * NB,), jnp.float32),
        pltpu.VMEM((NB,), jnp.float32),
    ],
    compiler_params=pltpu.CompilerParams(needs_layout_passes=False),
)(_sc_hist_body)


def _mask_body(t_ref, p_ref, o_ref):
    i = pl.program_id(0)
    t = t_ref[0, 0]
    p = p_ref[...]
    o = jnp.where(p >= t, p, 0.0)
    row = lax.broadcasted_iota(jnp.int32, (BLK, N), 0)
    col = lax.broadcasted_iota(jnp.int32, (BLK, N), 1)
    dm = col == row + i * BLK
    o_ref[...] = jnp.where(dm, jnp.maximum(o, 0.5), o)


_mask_call = pl.pallas_call(
    _mask_body,
    grid=(N // BLK,),
    in_specs=[
        pl.BlockSpec((1, 1), lambda i: (0, 0)),
        pl.BlockSpec((BLK, N), lambda i: (i, 0)),
    ],
    out_specs=pl.BlockSpec((BLK, N), lambda i: (i, 0)),
    out_shape=jax.ShapeDtypeStruct((N, N), jnp.float32),
)


def _params(lo, scale, mlo, mhi):
    vals = jnp.stack([lo, scale, mlo, mhi]).astype(jnp.float32)
    return jnp.repeat(vals, 16)


def kernel(static_feat, W1, b1, W2, b2):
    v1, v2 = _nodevec_call(static_feat, W1, b1.reshape(1, DIM),
                           W2, b2.reshape(1, DIM))
    p = _scores_call(v1, v2)
    pf = p.reshape(TOT)

    par1 = _params(jnp.float32(0.0), jnp.float32(NB),
                   jnp.float32(-1e30), jnp.float32(1e30))
    h1 = _sc_hist_call(pf, par1).reshape(NW, NB).astype(jnp.int32).sum(axis=0)
    suf1 = jnp.cumsum(h1[::-1])[::-1]
    j1 = jnp.sum((suf1 >= K_EDGES).astype(jnp.int32)) - 1
    above2 = suf1[j1] - h1[j1]
    lo2 = j1.astype(jnp.float32) / jnp.float32(NB)
    w1b = jnp.float32(1.0 / NB)

    par2 = _params(lo2, jnp.float32(NB * NB), lo2, lo2 + w1b)
    h2 = _sc_hist_call(pf, par2).reshape(NW, NB).astype(jnp.int32).sum(axis=0)
    suf2 = jnp.cumsum(h2[::-1])[::-1] + above2
    j2 = jnp.sum((suf2 >= K_EDGES).astype(jnp.int32)) - 1
    t = lo2 + j2.astype(jnp.float32) * jnp.float32(1.0 / (NB * NB))

    return _mask_call(t.reshape(1, 1), p)


# speedup vs baseline: 23.4173x; 1.2510x over previous
import functools

import jax
import jax.numpy as jnp
from jax import lax
from jax.experimental import pallas as pl
from jax.experimental.pallas import tpu as pltpu
from jax.experimental.pallas import tpu_sc as plsc

N = 4096
XD = 128
DIM = 64
K_EDGES = 65536
A1 = 0.1
A2 = 2.0

TOT = N * N
NB = 4096
NC = 2
NS = 16
NW = NC * NS
PER_W = TOT // NW
CH = 8192
BLK = 128

_PREC = lax.Precision.DEFAULT


def _nodevec_body(x_ref, w1_ref, b1_ref, w2_ref, b2_ref, v1_ref, v2_ref):
    x = x_ref[...]
    z1 = lax.dot_general(x, w1_ref[...], (((1,), (1,)), ((), ())),
                         precision=_PREC)
    z2 = lax.dot_general(x, w2_ref[...], (((1,), (1,)), ((), ())),
                         precision=_PREC)
    v1_ref[...] = jnp.tanh(A1 * (z1 + b1_ref[...]))
    v2_ref[...] = jnp.tanh(A1 * (z2 + b2_ref[...]))


_nodevec_call = pl.pallas_call(
    _nodevec_body,
    out_shape=(
        jax.ShapeDtypeStruct((N, DIM), jnp.float32),
        jax.ShapeDtypeStruct((N, DIM), jnp.float32),
    ),
)


def _scores_body(v1_ref, v2_ref, p_ref):
    s = lax.dot_general(v1_ref[...], v2_ref[...], (((1,), (1,)), ((), ())),
                        precision=_PREC)
    p_ref[...] = 1.0 / (1.0 + jnp.exp(-A2 * s))


_scores_call = pl.pallas_call(
    _scores_body,
    grid=(N // BLK,),
    in_specs=[
        pl.BlockSpec((BLK, DIM), lambda i: (i, 0)),
        pl.BlockSpec((N, DIM), lambda i: (0, 0)),
    ],
    out_specs=pl.BlockSpec((BLK, N), lambda i: (i, 0)),
    out_shape=jax.ShapeDtypeStruct((N, N), jnp.float32),
)


def _sc_hist_body(p_hbm, par_hbm, out_hbm, par_v, buf0_v, buf1_v, hist_v,
                  sem0, sem1):
    wid = lax.axis_index("s") * NC + lax.axis_index("c")
    base = wid * PER_W
    pltpu.sync_copy(par_hbm, par_v)
    lo = par_v[pl.ds(0, 16)]
    scale = par_v[pl.ds(16, 16)]
    mlo = par_v[pl.ds(32, 16)]
    mhi = par_v[pl.ds(48, 16)]

    def _zero(j, carry):
        for u in range(8):
            hist_v[pl.ds(j * 128 + u * 16, 16)] = jnp.zeros((16,), jnp.float32)
        return carry

    lax.fori_loop(0, NB // 128, _zero, 0)

    ones = jnp.ones((16,), jnp.float32)

    def _start(c, buf, sem):
        pltpu.async_copy(p_hbm.at[pl.ds(base + c * CH, CH)], buf, sem)

    def _wait(c, buf, sem):
        pltpu.make_async_copy(p_hbm.at[pl.ds(base + c * CH, CH)], buf,
                              sem).wait()

    def _process(buf):
        def _vec8(j, inner):
            for u in range(8):
                x = buf[pl.ds(j * 128 + u * 16, 16)]
                iv = jnp.clip(((x - lo) * scale).astype(jnp.int32), 0, NB - 1)
                m = (x >= mlo) & (x < mhi)
                plsc.addupdate_scatter(hist_v, [iv], ones, mask=m)
            return inner

        lax.fori_loop(0, CH // 128, _vec8, 0)

    NPAIR = PER_W // CH // 2
    _start(0, buf0_v, sem0)

    def _pair(g, carry):
        c0 = 2 * g
        _wait(c0, buf0_v, sem0)
        _start(c0 + 1, buf1_v, sem1)
        _process(buf0_v)
        _wait(c0 + 1, buf1_v, sem1)

        @pl.when(g + 1 < NPAIR)
        def _():
            _start(c0 + 2, buf0_v, sem0)

        _process(buf1_v)
        return carry

    lax.fori_loop(0, NPAIR, _pair, 0)
    pltpu.sync_copy(hist_v, out_hbm.at[pl.ds(wid * NB, NB)])


_sc_hist_call = functools.partial(
    pl.kernel,
    out_type=jax.ShapeDtypeStruct((NW * NB,), jnp.float32),
    mesh=plsc.VectorSubcoreMesh(core_axis_name="c", subcore_axis_name="s",
                                num_cores=NC, num_subcores=NS),
    scratch_types=[
        pltpu.VMEM((64,), jnp.float32),
        pltpu.VMEM((CH,), jnp.float32),
        pltpu.VMEM((CH,), jnp.float32),
        pltpu.VMEM((NB,), jnp.float32),
        pltpu.SemaphoreType.DMA,
        pltpu.SemaphoreType.DMA,
    ],
    compiler_params=pltpu.CompilerParams(needs_layout_passes=False),
)(_sc_hist_body)


def _mask_body(t_ref, p_ref, o_ref):
    i = pl.program_id(0)
    t = t_ref[0, 0]
    p = p_ref[...]
    o = jnp.where(p >= t, p, 0.0)
    row = lax.broadcasted_iota(jnp.int32, (BLK, N), 0)
    col = lax.broadcasted_iota(jnp.int32, (BLK, N), 1)
    dm = col == row + i * BLK
    o_ref[...] = jnp.where(dm, jnp.maximum(o, 0.5), o)


_mask_call = pl.pallas_call(
    _mask_body,
    grid=(N // BLK,),
    in_specs=[
        pl.BlockSpec((1, 1), lambda i: (0, 0)),
        pl.BlockSpec((BLK, N), lambda i: (i, 0)),
    ],
    out_specs=pl.BlockSpec((BLK, N), lambda i: (i, 0)),
    out_shape=jax.ShapeDtypeStruct((N, N), jnp.float32),
)


def _params(lo, scale, mlo, mhi):
    vals = jnp.stack([lo, scale, mlo, mhi]).astype(jnp.float32)
    return jnp.repeat(vals, 16)


def kernel(static_feat, W1, b1, W2, b2):
    v1, v2 = _nodevec_call(static_feat, W1, b1.reshape(1, DIM),
                           W2, b2.reshape(1, DIM))
    p = _scores_call(v1, v2)
    pf = p.reshape(TOT)

    par1 = _params(jnp.float32(0.0), jnp.float32(NB),
                   jnp.float32(-1e30), jnp.float32(1e30))
    h1 = _sc_hist_call(pf, par1).reshape(NW, NB).astype(jnp.int32).sum(axis=0)
    suf1 = jnp.cumsum(h1[::-1])[::-1]
    j1 = jnp.sum((suf1 >= K_EDGES).astype(jnp.int32)) - 1
    above2 = suf1[j1] - h1[j1]
    lo2 = j1.astype(jnp.float32) / jnp.float32(NB)
    w1b = jnp.float32(1.0 / NB)

    par2 = _params(lo2, jnp.float32(NB * NB), lo2, lo2 + w1b)
    h2 = _sc_hist_call(pf, par2).reshape(NW, NB).astype(jnp.int32).sum(axis=0)
    suf2 = jnp.cumsum(h2[::-1])[::-1] + above2
    j2 = jnp.sum((suf2 >= K_EDGES).astype(jnp.int32)) - 1
    t = lo2 + j2.astype(jnp.float32) * jnp.float32(1.0 / (NB * NB))

    return _mask_call(t.reshape(1, 1), p)


# speedup vs baseline: 24.7290x; 1.0560x over previous
import functools

import jax
import jax.numpy as jnp
from jax import lax
from jax.experimental import pallas as pl
from jax.experimental.pallas import tpu as pltpu
from jax.experimental.pallas import tpu_sc as plsc

N = 4096
XD = 128
DIM = 64
K_EDGES = 65536
A1 = 0.1
A2 = 2.0

TOT = N * N
NB = 4096
NC = 2
NS = 16
NW = NC * NS
PER_W = TOT // NW
CH = 8192
BLK = 128

_PREC = lax.Precision.DEFAULT


def _nodevec_body(x_ref, w1_ref, b1_ref, w2_ref, b2_ref, v1_ref, v2_ref):
    x = x_ref[...]
    z1 = lax.dot_general(x, w1_ref[...], (((1,), (1,)), ((), ())),
                         precision=_PREC)
    z2 = lax.dot_general(x, w2_ref[...], (((1,), (1,)), ((), ())),
                         precision=_PREC)
    v1_ref[...] = jnp.tanh(A1 * (z1 + b1_ref[...]))
    v2_ref[...] = jnp.tanh(A1 * (z2 + b2_ref[...]))


_nodevec_call = pl.pallas_call(
    _nodevec_body,
    out_shape=(
        jax.ShapeDtypeStruct((N, DIM), jnp.float32),
        jax.ShapeDtypeStruct((N, DIM), jnp.float32),
    ),
)


def _scores_body(v1_ref, v2_ref, p_ref):
    s = lax.dot_general(v1_ref[...], v2_ref[...], (((1,), (1,)), ((), ())),
                        precision=_PREC)
    p = 1.0 / (1.0 + jnp.exp(-A2 * s))
    p_ref[...] = p.reshape(BLK * N)


_scores_call = pl.pallas_call(
    _scores_body,
    grid=(N // BLK,),
    in_specs=[
        pl.BlockSpec((BLK, DIM), lambda i: (i, 0)),
        pl.BlockSpec((N, DIM), lambda i: (0, 0)),
    ],
    out_specs=pl.BlockSpec((BLK * N,), lambda i: (i,)),
    out_shape=jax.ShapeDtypeStruct((TOT,), jnp.float32),
)


def _sc_hist_body(p_hbm, par_hbm, out_hbm, par_v, buf0_v, buf1_v, hist_v,
                  sem0, sem1):
    wid = lax.axis_index("s") * NC + lax.axis_index("c")
    base = wid * PER_W
    pltpu.sync_copy(par_hbm, par_v)
    lo = par_v[pl.ds(0, 16)]
    scale = par_v[pl.ds(16, 16)]
    mlo = par_v[pl.ds(32, 16)]
    mhi = par_v[pl.ds(48, 16)]

    def _zero(j, carry):
        for u in range(8):
            hist_v[pl.ds(j * 128 + u * 16, 16)] = jnp.zeros((16,), jnp.float32)
        return carry

    lax.fori_loop(0, NB // 128, _zero, 0)

    ones = jnp.ones((16,), jnp.float32)

    def _start(c, buf, sem):
        pltpu.async_copy(p_hbm.at[pl.ds(base + c * CH, CH)], buf, sem)

    def _wait(c, buf, sem):
        pltpu.make_async_copy(p_hbm.at[pl.ds(base + c * CH, CH)], buf,
                              sem).wait()

    def _process(buf):
        def _vec8(j, inner):
            for u in range(8):
                x = buf[pl.ds(j * 128 + u * 16, 16)]
                iv = jnp.clip(((x - lo) * scale).astype(jnp.int32), 0, NB - 1)
                m = (x >= mlo) & (x < mhi)
                plsc.addupdate_scatter(hist_v, [iv], ones, mask=m)
            return inner

        lax.fori_loop(0, CH // 128, _vec8, 0)

    NPAIR = PER_W // CH // 2
    _start(0, buf0_v, sem0)

    def _pair(g, carry):
        c0 = 2 * g
        _wait(c0, buf0_v, sem0)
        _start(c0 + 1, buf1_v, sem1)
        _process(buf0_v)
        _wait(c0 + 1, buf1_v, sem1)

        @pl.when(g + 1 < NPAIR)
        def _():
            _start(c0 + 2, buf0_v, sem0)

        _process(buf1_v)
        return carry

    lax.fori_loop(0, NPAIR, _pair, 0)
    pltpu.sync_copy(hist_v, out_hbm.at[pl.ds(wid * NB, NB)])


_sc_hist_call = functools.partial(
    pl.kernel,
    out_type=jax.ShapeDtypeStruct((NW * NB,), jnp.float32),
    mesh=plsc.VectorSubcoreMesh(core_axis_name="c", subcore_axis_name="s",
                                num_cores=NC, num_subcores=NS),
    scratch_types=[
        pltpu.VMEM((64,), jnp.float32),
        pltpu.VMEM((CH,), jnp.float32),
        pltpu.VMEM((CH,), jnp.float32),
        pltpu.VMEM((NB,), jnp.float32),
        pltpu.SemaphoreType.DMA,
        pltpu.SemaphoreType.DMA,
    ],
    compiler_params=pltpu.CompilerParams(needs_layout_passes=False),
)(_sc_hist_body)


def _mask_body(t_ref, v1_ref, v2_ref, o_ref):
    i = pl.program_id(0)
    t = t_ref[0, 0]
    s = lax.dot_general(v1_ref[...], v2_ref[...], (((1,), (1,)), ((), ())),
                        precision=_PREC)
    p = 1.0 / (1.0 + jnp.exp(-A2 * s))
    o = jnp.where(p >= t, p, 0.0)
    row = lax.broadcasted_iota(jnp.int32, (BLK, N), 0)
    col = lax.broadcasted_iota(jnp.int32, (BLK, N), 1)
    dm = col == row + i * BLK
    o_ref[...] = jnp.where(dm, jnp.maximum(o, 0.5), o)


_mask_call = pl.pallas_call(
    _mask_body,
    grid=(N // BLK,),
    in_specs=[
        pl.BlockSpec((1, 1), lambda i: (0, 0)),
        pl.BlockSpec((BLK, DIM), lambda i: (i, 0)),
        pl.BlockSpec((N, DIM), lambda i: (0, 0)),
    ],
    out_specs=pl.BlockSpec((BLK, N), lambda i: (i, 0)),
    out_shape=jax.ShapeDtypeStruct((N, N), jnp.float32),
)


def _params(lo, scale, mlo, mhi):
    vals = jnp.stack([lo, scale, mlo, mhi]).astype(jnp.float32)
    return jnp.repeat(vals, 16)


def kernel(static_feat, W1, b1, W2, b2):
    v1, v2 = _nodevec_call(static_feat, W1, b1.reshape(1, DIM),
                           W2, b2.reshape(1, DIM))
    pf = _scores_call(v1, v2)

    par1 = _params(jnp.float32(0.0), jnp.float32(NB),
                   jnp.float32(-1e30), jnp.float32(1e30))
    h1 = _sc_hist_call(pf, par1).reshape(NW, NB).astype(jnp.int32).sum(axis=0)
    suf1 = jnp.cumsum(h1[::-1])[::-1]
    j1 = jnp.sum((suf1 >= K_EDGES).astype(jnp.int32)) - 1
    above2 = suf1[j1] - h1[j1]
    lo2 = j1.astype(jnp.float32) / jnp.float32(NB)
    w1b = jnp.float32(1.0 / NB)

    par2 = _params(lo2, jnp.float32(NB * NB), lo2, lo2 + w1b)
    h2 = _sc_hist_call(pf, par2).reshape(NW, NB).astype(jnp.int32).sum(axis=0)
    suf2 = jnp.cumsum(h2[::-1])[::-1] + above2
    j2 = jnp.sum((suf2 >= K_EDGES).astype(jnp.int32)) - 1
    t = lo2 + j2.astype(jnp.float32) * jnp.float32(1.0 / (NB * NB))

    return _mask_call(t.reshape(1, 1), v1, v2)


# speedup vs baseline: 87.1637x; 3.5248x over previous
import functools

import jax
import jax.numpy as jnp
from jax import lax
from jax.experimental import pallas as pl
from jax.experimental.pallas import tpu as pltpu
from jax.experimental.pallas import tpu_sc as plsc

N = 4096
XD = 128
DIM = 64
K_EDGES = 65536
A1 = 0.1
A2 = 2.0

TOT = N * N
NB = 4096
NC = 2
NS = 16
NW = NC * NS
PER_W = TOT // NW
CH = 8192
BLK = 128

_PREC = lax.Precision.DEFAULT


def _nodevec_body(x_ref, w1_ref, b1_ref, w2_ref, b2_ref, v1_ref, v2_ref):
    x = x_ref[...]
    z1 = lax.dot_general(x, w1_ref[...], (((1,), (1,)), ((), ())),
                         precision=_PREC)
    z2 = lax.dot_general(x, w2_ref[...], (((1,), (1,)), ((), ())),
                         precision=_PREC)
    v1_ref[...] = jnp.tanh(A1 * (z1 + b1_ref[...]))
    v2_ref[...] = jnp.tanh(A1 * (z2 + b2_ref[...]))


_nodevec_call = pl.pallas_call(
    _nodevec_body,
    out_shape=(
        jax.ShapeDtypeStruct((N, DIM), jnp.float32),
        jax.ShapeDtypeStruct((N, DIM), jnp.float32),
    ),
)


def _scores_body(v1_ref, v2_ref, p_ref):
    s = lax.dot_general(v1_ref[...], v2_ref[...], (((1,), (1,)), ((), ())),
                        precision=_PREC)
    p = 1.0 / (1.0 + jnp.exp(-A2 * s))
    p_ref[...] = p.reshape(BLK * N)


_scores_call = pl.pallas_call(
    _scores_body,
    grid=(N // BLK,),
    in_specs=[
        pl.BlockSpec((BLK, DIM), lambda i: (i, 0)),
        pl.BlockSpec((N, DIM), lambda i: (0, 0)),
    ],
    out_specs=pl.BlockSpec((BLK * N,), lambda i: (i,)),
    out_shape=jax.ShapeDtypeStruct((TOT,), jnp.float32),
)


def _sc_hist_body(p_hbm, par_hbm, out_hbm, par_v, buf0_v, buf1_v, hist_v,
                  sem0, sem1):
    wid = lax.axis_index("s") * NC + lax.axis_index("c")
    base = wid * PER_W
    pltpu.sync_copy(par_hbm, par_v)
    lo = par_v[pl.ds(0, 16)]
    scale = par_v[pl.ds(16, 16)]
    mlo = par_v[pl.ds(32, 16)]
    mhi = par_v[pl.ds(48, 16)]

    @plsc.parallel_loop(0, NB, 16, unroll=8)
    def _zero(off):
        hist_v[pl.ds(off, 16)] = jnp.zeros((16,), jnp.float32)

    ones = jnp.ones((16,), jnp.float32)

    def _start(c, buf, sem):
        pltpu.async_copy(p_hbm.at[pl.ds(base + c * CH, CH)], buf, sem)

    def _wait(c, buf, sem):
        pltpu.make_async_copy(p_hbm.at[pl.ds(base + c * CH, CH)], buf,
                              sem).wait()

    def _process(buf):
        @plsc.parallel_loop(0, CH, 16, unroll=8)
        def _vec(off):
            x = buf[pl.ds(off, 16)]
            iv = jnp.clip(((x - lo) * scale).astype(jnp.int32), 0, NB - 1)
            m = (x >= mlo) & (x < mhi)
            plsc.addupdate_scatter(hist_v, [iv], ones, mask=m)

    NPAIR = PER_W // CH // 2
    _start(0, buf0_v, sem0)

    def _pair(g, carry):
        c0 = 2 * g
        _wait(c0, buf0_v, sem0)
        _start(c0 + 1, buf1_v, sem1)
        _process(buf0_v)
        _wait(c0 + 1, buf1_v, sem1)

        @pl.when(g + 1 < NPAIR)
        def _():
            _start(c0 + 2, buf0_v, sem0)

        _process(buf1_v)
        return carry

    lax.fori_loop(0, NPAIR, _pair, 0)
    pltpu.sync_copy(hist_v, out_hbm.at[pl.ds(wid * NB, NB)])


_sc_hist_call = functools.partial(
    pl.kernel,
    out_type=jax.ShapeDtypeStruct((NW * NB,), jnp.float32),
    mesh=plsc.VectorSubcoreMesh(core_axis_name="c", subcore_axis_name="s",
                                num_cores=NC, num_subcores=NS),
    scratch_types=[
        pltpu.VMEM((64,), jnp.float32),
        pltpu.VMEM((CH,), jnp.float32),
        pltpu.VMEM((CH,), jnp.float32),
        pltpu.VMEM((NB,), jnp.float32),
        pltpu.SemaphoreType.DMA,
        pltpu.SemaphoreType.DMA,
    ],
    compiler_params=pltpu.CompilerParams(needs_layout_passes=False),
)(_sc_hist_body)


def _mask_body(t_ref, v1_ref, v2_ref, o_ref):
    i = pl.program_id(0)
    t = t_ref[0, 0]
    s = lax.dot_general(v1_ref[...], v2_ref[...], (((1,), (1,)), ((), ())),
                        precision=_PREC)
    p = 1.0 / (1.0 + jnp.exp(-A2 * s))
    o = jnp.where(p >= t, p, 0.0)
    row = lax.broadcasted_iota(jnp.int32, (BLK, N), 0)
    col = lax.broadcasted_iota(jnp.int32, (BLK, N), 1)
    dm = col == row + i * BLK
    o_ref[...] = jnp.where(dm, jnp.maximum(o, 0.5), o)


_mask_call = pl.pallas_call(
    _mask_body,
    grid=(N // BLK,),
    in_specs=[
        pl.BlockSpec((1, 1), lambda i: (0, 0)),
        pl.BlockSpec((BLK, DIM), lambda i: (i, 0)),
        pl.BlockSpec((N, DIM), lambda i: (0, 0)),
    ],
    out_specs=pl.BlockSpec((BLK, N), lambda i: (i, 0)),
    out_shape=jax.ShapeDtypeStruct((N, N), jnp.float32),
)


def _params(lo, scale, mlo, mhi):
    vals = jnp.stack([lo, scale, mlo, mhi]).astype(jnp.float32)
    return jnp.repeat(vals, 16)


def kernel(static_feat, W1, b1, W2, b2):
    v1, v2 = _nodevec_call(static_feat, W1, b1.reshape(1, DIM),
                           W2, b2.reshape(1, DIM))
    pf = _scores_call(v1, v2)

    par1 = _params(jnp.float32(0.0), jnp.float32(NB),
                   jnp.float32(-1e30), jnp.float32(1e30))
    h1 = _sc_hist_call(pf, par1).reshape(NW, NB).astype(jnp.int32).sum(axis=0)
    suf1 = jnp.cumsum(h1[::-1])[::-1]
    j1 = jnp.sum((suf1 >= K_EDGES).astype(jnp.int32)) - 1
    above2 = suf1[j1] - h1[j1]
    lo2 = j1.astype(jnp.float32) / jnp.float32(NB)
    w1b = jnp.float32(1.0 / NB)

    par2 = _params(lo2, jnp.float32(NB * NB), lo2, lo2 + w1b)
    h2 = _sc_hist_call(pf, par2).reshape(NW, NB).astype(jnp.int32).sum(axis=0)
    suf2 = jnp.cumsum(h2[::-1])[::-1] + above2
    j2 = jnp.sum((suf2 >= K_EDGES).astype(jnp.int32)) - 1
    t = lo2 + j2.astype(jnp.float32) * jnp.float32(1.0 / (NB * NB))

    return _mask_call(t.reshape(1, 1), v1, v2)
